# Initial kernel scaffold; baseline (speedup 1.0000x reference)
import jax
import jax.numpy as jnp
from jax.experimental import pallas as pl


def kernel(x, pos_table):
    raise NotImplementedError("write your pallas kernel here")


# speedup vs baseline: 3.2262x; 3.2262x over previous
import jax
import jax.numpy as jnp
from jax.experimental import pallas as pl

BS = 256


def _add_body(x_ref, t_ref, o_ref):
    o_ref[...] = x_ref[...] + t_ref[...][None, :, :]


def kernel(x, pos_table):
    b, s, d = x.shape
    grid = (s // BS,)
    return pl.pallas_call(
        _add_body,
        grid=grid,
        in_specs=[
            pl.BlockSpec((b, BS, d), lambda g: (0, g, 0)),
            pl.BlockSpec((BS, d), lambda g: (g, 0)),
        ],
        out_specs=pl.BlockSpec((b, BS, d), lambda g: (0, g, 0)),
        out_shape=jax.ShapeDtypeStruct((b, s, d), x.dtype),
    )(x, pos_table[:s])
